# Initial kernel scaffold; baseline (speedup 1.0000x reference)
import jax
import jax.numpy as jnp
from jax.experimental import pallas as pl


def kernel(x, edge_index, beta1, beta2, beta3):
    raise NotImplementedError("write your pallas kernel here")


# speedup vs baseline: 2.0998x; 2.0998x over previous
import functools

import jax
import jax.numpy as jnp
from jax import lax
from jax.experimental import pallas as pl
from jax.experimental.pallas import tpu as pltpu
from jax.experimental.pallas import tpu_sc as plsc

N = 10000
D = 128
NC = 2
NS = 16
NW = NC * NS
NPAD = 10240
RPT = NPAD // NS
EB = 128
BR = 512


def _prep_first_body(x_ref, y_ref, n_ref):
    xb = x_ref[...]
    n = jnp.sqrt(jnp.sum(xb * xb, axis=1, keepdims=True))
    y_ref[...] = xb / jnp.clip(n, 1e-12, None)
    n_ref[...] = n


def _prep_mid_body(p0_ref, p1_ref, d0_ref, d1_ref, y_ref, n_ref):
    den = jnp.clip(d0_ref[...] + d1_ref[...], 1e-16, None)
    h = (p0_ref[...] + p1_ref[...]) / den
    n = jnp.sqrt(jnp.sum(h * h, axis=1, keepdims=True))
    y_ref[...] = h / jnp.clip(n, 1e-12, None)
    n_ref[...] = n


def _prep_last_body(p0_ref, p1_ref, d0_ref, d1_ref, h_ref):
    den = jnp.clip(d0_ref[...] + d1_ref[...], 1e-16, None)
    h_ref[...] = (p0_ref[...] + p1_ref[...]) / den


_ROW = pl.BlockSpec((BR, D), lambda i: (i, 0))
_COL = pl.BlockSpec((BR, 1), lambda i: (i, 0))

_prep_first = pl.pallas_call(
    _prep_first_body,
    grid=(NPAD // BR,),
    in_specs=[_ROW],
    out_specs=[_ROW, _COL],
    out_shape=[jax.ShapeDtypeStruct((NPAD, D), jnp.float32),
               jax.ShapeDtypeStruct((NPAD, 1), jnp.float32)],
)

_prep_mid = pl.pallas_call(
    _prep_mid_body,
    grid=(NPAD // BR,),
    in_specs=[_ROW, _ROW, _COL, _COL],
    out_specs=[_ROW, _COL],
    out_shape=[jax.ShapeDtypeStruct((NPAD, D), jnp.float32),
               jax.ShapeDtypeStruct((NPAD, 1), jnp.float32)],
)

_prep_last = pl.pallas_call(
    _prep_last_body,
    grid=(NPAD // BR,),
    in_specs=[_ROW, _ROW, _COL, _COL],
    out_specs=[_ROW],
    out_shape=[jax.ShapeDtypeStruct((NPAD, D), jnp.float32)],
)


def _sc_edge_body(nb, y_hbm, nrm_hbm, src_hbm, dst_hbm, beta_hbm,
                  out_hbm, den_hbm,
                  out_sh, den_sh, sidx, didx, ysrc, ydst, nsrc, wbuf,
                  betav, sem0, sem1, sem2):
    c = lax.axis_index("c")
    s = lax.axis_index("s")
    wid = c * NS + s

    pltpu.sync_copy(beta_hbm, betav)

    z16 = jnp.zeros((16,), jnp.float32)

    def _zrow(i, carry):
        def _zcol(k, cc):
            ysrc[i, pl.ds(k * 16, 16)] = z16
            return cc
        return lax.fori_loop(0, D // 16, _zcol, carry)

    lax.fori_loop(0, EB, _zrow, 0)

    t0 = s * RPT
    for r in range(RPT // EB):
        pltpu.sync_copy(ysrc, out_sh.at[pl.ds(t0 + r * EB, EB)])
        pltpu.sync_copy(ysrc.at[0], den_sh.at[pl.ds(t0 + r * EB, EB)])
    plsc.subcore_barrier()

    lanes = lax.iota(jnp.int32, 16)
    bv = betav[...]
    base_e = wid * (nb * EB)

    def _block(b, carry):
        off = base_e + b * EB
        pltpu.sync_copy(src_hbm.at[pl.ds(off, EB)], sidx)
        pltpu.sync_copy(dst_hbm.at[pl.ds(off, EB)], didx)
        cp0 = pltpu.async_copy(y_hbm.at[sidx], ysrc, sem0)
        cp1 = pltpu.async_copy(y_hbm.at[didx], ydst, sem1)
        cp2 = pltpu.async_copy(nrm_hbm.at[sidx], nsrc, sem2)
        cp0.wait()
        cp1.wait()
        cp2.wait()

        def _group(g, gc):
            rows = g * 16 + lanes

            def _dot(dd, acc):
                for k in range(8):
                    col = lanes * 0 + (dd * 8 + k)
                    a = plsc.load_gather(ysrc, [rows, col])
                    bb = plsc.load_gather(ydst, [rows, col])
                    acc = acc + a * bb
                return acc

            acc = lax.fori_loop(0, D // 8, _dot, jnp.zeros((16,), jnp.float32))
            w = jnp.exp(acc * bv)
            ns_v = plsc.load_gather(nsrc, [rows])
            plsc.store_scatter(wbuf, [rows], w)
            scale = w * ns_v

            def _scale(dd, sc2):
                for k in range(8):
                    col = lanes * 0 + (dd * 8 + k)
                    a = plsc.load_gather(ysrc, [rows, col])
                    plsc.store_scatter(ysrc, [rows, col], a * scale)
                return sc2

            lax.fori_loop(0, D // 8, _scale, 0)
            return gc

        lax.fori_loop(0, EB // 16, _group, 0)
        pltpu.sync_copy(ysrc, out_sh.at[didx], add=True)
        pltpu.sync_copy(wbuf, den_sh.at[didx], add=True)
        return carry

    lax.fori_loop(0, nb, _block, 0)
    plsc.subcore_barrier()

    pltpu.sync_copy(out_sh.at[pl.ds(t0, RPT)], out_hbm.at[c, pl.ds(t0, RPT)])
    pltpu.sync_copy(den_sh.at[pl.ds(t0, RPT)], den_hbm.at[c, pl.ds(t0, RPT)])


@functools.lru_cache(maxsize=None)
def _make_sc_edge(nb):
    mesh = plsc.VectorSubcoreMesh(core_axis_name="c", subcore_axis_name="s",
                                  num_cores=NC, num_subcores=NS)
    return pl.kernel(
        functools.partial(_sc_edge_body, nb),
        out_type=[jax.ShapeDtypeStruct((NC, NPAD, D), jnp.float32),
                  jax.ShapeDtypeStruct((NC, NPAD), jnp.float32)],
        mesh=mesh,
        compiler_params=pltpu.CompilerParams(needs_layout_passes=False),
        scratch_types=[
            pltpu.VMEM_SHARED((NPAD, D), jnp.float32),
            pltpu.VMEM_SHARED((NPAD,), jnp.float32),
            pltpu.VMEM((EB,), jnp.int32),
            pltpu.VMEM((EB,), jnp.int32),
            pltpu.VMEM((EB, D), jnp.float32),
            pltpu.VMEM((EB, D), jnp.float32),
            pltpu.VMEM((EB,), jnp.float32),
            pltpu.VMEM((EB,), jnp.float32),
            pltpu.VMEM((16,), jnp.float32),
            pltpu.SemaphoreType.DMA,
            pltpu.SemaphoreType.DMA,
            pltpu.SemaphoreType.DMA,
        ],
    )


def kernel(x, edge_index, beta1, beta2, beta3):
    loops = jnp.arange(N, dtype=jnp.int32)
    src = jnp.concatenate([edge_index[0].astype(jnp.int32), loops])
    dst = jnp.concatenate([edge_index[1].astype(jnp.int32), loops])
    e_tot = src.shape[0]
    nb = -(-e_tot // (NW * EB))
    epad = nb * EB * NW
    pad = epad - e_tot
    src = jnp.concatenate([src, jnp.full((pad,), N, jnp.int32)])
    dst = jnp.concatenate([dst, jnp.full((pad,), N, jnp.int32)])
    xp = jnp.zeros((NPAD, D), jnp.float32).at[:N].set(x)

    sc_edge = _make_sc_edge(nb)
    y, nrm = _prep_first(xp)
    for i, beta in enumerate((beta1, beta2, beta3)):
        bvec = jnp.full((16,), beta, jnp.float32)
        outp, denp = sc_edge(y, nrm.reshape(NPAD), src, dst, bvec)
        d0 = denp[0].reshape(NPAD, 1)
        d1 = denp[1].reshape(NPAD, 1)
        if i < 2:
            y, nrm = _prep_mid(outp[0], outp[1], d0, d1)
        else:
            h = _prep_last(outp[0], outp[1], d0, d1)[0]
    return h[:N]
